# baseline (speedup 1.0000x reference)
import functools

import jax
import jax.numpy as jnp
from jax import lax
from jax.experimental import pallas as pl
from jax.experimental.pallas import tpu as pltpu
from jax.experimental.pallas import tpu_sc as plsc

N = 10000
E = 320000
D1 = 128
D2 = 48
NC = 2
NS = 16
NW = NC * NS
EPW = E // NW
K = 80
NWIN = EPW // K
NP = 10240
RPT = NP // NS

_MESH = dict(core_axis_name="c", subcore_axis_name="s")


_NSEM = 4


@functools.partial(
    pl.kernel,
    out_type=jax.ShapeDtypeStruct((NC * NP,), jnp.float32),
    mesh=plsc.VectorSubcoreMesh(**_MESH),
    scratch_types=[
        pltpu.VMEM((NWIN, K), jnp.int32),
        pltpu.VMEM((K,), jnp.float32),
        pltpu.VMEM((RPT,), jnp.float32),
        [pltpu.SemaphoreType.DMA] * _NSEM,
        [pltpu.SemaphoreType.DMA] * _NSEM,
        pltpu.VMEM_SHARED((NP,), jnp.float32),
    ],
)
def _sc_degree(dst_hbm, out_hbm, didx, ones_v, zeros_v, isems, sems, deg_sh):
    cid = lax.axis_index("c")
    sid = lax.axis_index("s")
    wid = cid * NS + sid

    def iload(j, _):
        for b in range(_NSEM):
            jw = j * _NSEM + b
            pltpu.async_copy(dst_hbm.at[pl.ds(wid * EPW + jw * K, K)],
                             didx.at[jw], isems[b])
        return 0
    lax.fori_loop(0, NWIN // _NSEM, iload, 0)
    for b in range(NWIN % _NSEM):
        jw = NWIN - 1 - b
        pltpu.async_copy(dst_hbm.at[pl.ds(wid * EPW + jw * K, K)],
                         didx.at[jw], isems[b])

    def fill(i, _):
        zeros_v[pl.ds(i * 16, 16)] = jnp.zeros((16,), jnp.float32)
        return 0
    lax.fori_loop(0, RPT // 16, fill, 0)
    for i in range(K // 16):
        ones_v[pl.ds(i * 16, 16)] = jnp.full((16,), 1.0, jnp.float32)

    pltpu.sync_copy(zeros_v, deg_sh.at[pl.ds(sid * RPT, RPT)])

    def idrain(j, _):
        for b in range(_NSEM):
            pltpu.make_async_copy(dst_hbm.at[pl.ds(0, K)], didx.at[0],
                                  isems[b]).wait()
        return 0
    lax.fori_loop(0, NWIN // _NSEM, idrain, 0)
    for b in range(NWIN % _NSEM):
        pltpu.make_async_copy(dst_hbm.at[pl.ds(0, K)], didx.at[0],
                              isems[b]).wait()
    plsc.subcore_barrier()

    def body(j, _):
        for b in range(_NSEM):
            pltpu.async_copy(ones_v, deg_sh.at[didx.at[j * _NSEM + b]],
                             sems[b], add=True)
        return 0
    lax.fori_loop(0, NWIN // _NSEM, body, 0)
    for b in range(NWIN % _NSEM):
        pltpu.async_copy(ones_v, deg_sh.at[didx.at[NWIN - 1 - b]],
                         sems[b], add=True)

    def drain(j, _):
        for b in range(_NSEM):
            pltpu.make_async_copy(ones_v, deg_sh.at[didx.at[0]],
                                  sems[b]).wait()
        return 0
    lax.fori_loop(0, NWIN // _NSEM, drain, 0)
    for b in range(NWIN % _NSEM):
        pltpu.make_async_copy(ones_v, deg_sh.at[didx.at[0]], sems[b]).wait()

    plsc.subcore_barrier()
    pltpu.sync_copy(deg_sh.at[pl.ds(sid * RPT, RPT)],
                    out_hbm.at[pl.ds(cid * NP + sid * RPT, RPT)])


def _make_sc_aggregate(D, NBUF, zero_buf, stage_table):
    scratch = [
        pltpu.VMEM((EPW,), jnp.int32),
        pltpu.VMEM((NBUF, K), jnp.int32),
        [pltpu.VMEM((K, D), jnp.float32)] * NBUF,
        [pltpu.SemaphoreType.DMA] * NBUF,
        [pltpu.SemaphoreType.DMA] * NBUF,
        [pltpu.SemaphoreType.DMA] * NBUF,
        pltpu.VMEM_SHARED((NP, D), jnp.float32),
    ]
    if stage_table:
        scratch.append(pltpu.VMEM_SHARED((NP, D), jnp.float32))

    @functools.partial(
        pl.kernel,
        out_type=jax.ShapeDtypeStruct((NC * NP, D), jnp.float32),
        mesh=plsc.VectorSubcoreMesh(**_MESH),
        compiler_params=pltpu.CompilerParams(use_tc_tiling_on_sc=(D % 128 == 0)),
        scratch_types=scratch,
    )
    def agg(xws_hbm, src_hbm, dst_hbm, out_hbm, sidx, didx, rows, gsem,
            ssem, isem, acc_sh, *maybe_table):
        cid = lax.axis_index("c")
        sid = lax.axis_index("s")
        wid = cid * NS + sid
        table = maybe_table[0] if stage_table else xws_hbm

        def istart(j, b):
            pltpu.async_copy(dst_hbm.at[pl.ds(wid * EPW + j * K, K)],
                             didx.at[b], isem[b])

        def iwait(b):
            pltpu.make_async_copy(dst_hbm.at[pl.ds(0, K)], didx.at[b],
                                  isem[b]).wait()

        def gstart(j, b):
            pltpu.async_copy(table.at[sidx.at[pl.ds(j * K, K)]],
                             rows[b], gsem[b])

        def gwait(b):
            pltpu.make_async_copy(table.at[sidx.at[pl.ds(0, K)]],
                                  rows[b], gsem[b]).wait()

        def sstart(b):
            pltpu.async_copy(rows[b], acc_sh.at[didx.at[b]], ssem[b],
                             add=True)

        def swait(b):
            pltpu.make_async_copy(rows[b], acc_sh.at[didx.at[b]],
                                  ssem[b]).wait()

        pltpu.sync_copy(src_hbm.at[pl.ds(wid * EPW, EPW)], sidx)
        for b in range(NBUF):
            istart(b, b)
        if stage_table:
            @pl.when(sid < 10)
            def _():
                pltpu.sync_copy(xws_hbm.at[pl.ds(sid * 1000, 1000)],
                                maybe_table[0].at[pl.ds(sid * 1000, 1000)])
        else:
            for b in range(NBUF):
                if b != zero_buf:
                    gstart(b, b)

        def fill_row(i, _):
            def fill_lane(c, _):
                rows[zero_buf][i, pl.ds(c * 16, 16)] = (
                    jnp.zeros((16,), jnp.float32))
                return 0
            lax.fori_loop(0, D // 16, fill_lane, 0)
            return 0
        lax.fori_loop(0, K, fill_row, 0)
        for i in range(RPT // K):
            pltpu.sync_copy(rows[zero_buf],
                            acc_sh.at[pl.ds(sid * RPT + i * K, K)])
        plsc.subcore_barrier()
        if stage_table:
            for b in range(NBUF):
                if b != zero_buf:
                    gstart(b, b)
        gstart(zero_buf, zero_buf)

        def group(g, _):
            for b in range(NBUF):
                j = g * NBUF + b

                @pl.when(j < NWIN)
                def _():
                    gwait(b)
                    iwait(b)
                    sstart(b)

                    @pl.when(j + NBUF < NWIN)
                    def _():
                        swait(b)
                        gstart(j + NBUF, b)
                        istart(j + NBUF, b)
            return 0
        lax.fori_loop(0, (NWIN + NBUF - 1) // NBUF, group, 0)

        for b in range(NBUF):
            swait(b)

        plsc.subcore_barrier()
        pltpu.sync_copy(acc_sh.at[pl.ds(sid * RPT, RPT)],
                        out_hbm.at[pl.ds(cid * NP + sid * RPT, RPT)])
    return agg


_sc_agg128 = _make_sc_aggregate(D1, 3, 2, False)
_sc_agg48 = _make_sc_aggregate(D2, 4, 3, False)


_DR = NP // 128


def _node_blocks():
    out = []
    for r in range((N + 127) // 128):
        n = min(128, N - r * 128)
        out.append((slice(r * 128, r * 128 + n), n, r))
    return out


def _tc_pre_body(degp_ref, x_ref, w1_ref, xws_ref, dinv_ref):
    degp = degp_ref[...]
    deg = degp[:_DR] + degp[_DR:] + 1.0
    dt = lax.rsqrt(deg).T
    xw = jnp.dot(x_ref[...], w1_ref[...], preferred_element_type=jnp.float32)
    for rows, n, r in _node_blocks():
        col = dt[:n, r:r + 1]
        xws_ref[rows, :] = xw[rows, :] * col
        dinv_ref[rows, :] = jnp.broadcast_to(col, (n, D1))


_tc_pre = pl.pallas_call(
    _tc_pre_body,
    out_shape=(jax.ShapeDtypeStruct((N, D1), jnp.float32),
               jax.ShapeDtypeStruct((N, D1), jnp.float32)),
)


def _tc_mid_body(acc, xws1, dinv, b1, gamma, beta, w2p, xws2_ref):
    dv = dinv[...]
    h = (acc[:N] + acc[NP:NP + N] + xws1[...]) * dv + b1[...]
    mean = jnp.mean(h, axis=0, keepdims=True)
    c = h - mean
    var = jnp.mean(c * c, axis=0, keepdims=True)
    hn = c * lax.rsqrt(var + 1e-5) * gamma[...] + beta[...]
    hr = jnp.maximum(hn, 0.0)
    xw2 = jnp.dot(hr, w2p[...], preferred_element_type=jnp.float32)
    xws2_ref[...] = xw2 * dv[:, :D2]


_tc_mid = pl.pallas_call(
    _tc_mid_body,
    out_shape=jax.ShapeDtypeStruct((N, D2), jnp.float32),
)


def _tc_post_body(acc, xws2, degp, b2, out_ref):
    dp = degp[...]
    dt = lax.rsqrt(dp[:_DR] + dp[_DR:] + 1.0).T
    s = acc[:N] + acc[NP:NP + N] + xws2[...]
    for rows, n, r in _node_blocks():
        col = dt[:n, r:r + 1]
        out_ref[:, rows] = (s[rows, :NCLS] * col + b2[...]).T


NCLS = 40

_tc_post = pl.pallas_call(
    _tc_post_body,
    out_shape=jax.ShapeDtypeStruct((NCLS, N), jnp.float32),
)


def kernel(x, edge_index, W1, b1, bn_gamma, bn_beta, W2, b2):
    src = edge_index[0]
    dst = edge_index[1]

    degp = _sc_degree(dst)
    degp = degp.reshape(2 * _DR, 128)

    xws1, dinv = _tc_pre(degp, x, W1)

    acc1 = _sc_agg128(xws1, src, dst)

    w2p = jnp.pad(W2, ((0, 0), (0, D2 - W2.shape[1])))
    xws2 = _tc_mid(acc1, xws1, dinv,
                   b1.reshape(1, D1), bn_gamma.reshape(1, D1),
                   bn_beta.reshape(1, D1), w2p)

    acc2 = _sc_agg48(xws2, src, dst)

    return _tc_post(acc2, xws2, degp, b2.reshape(1, NCLS)).T


# speedup vs baseline: 1.0589x; 1.0589x over previous
import functools

import jax
import jax.numpy as jnp
from jax import lax
from jax.experimental import pallas as pl
from jax.experimental.pallas import tpu as pltpu
from jax.experimental.pallas import tpu_sc as plsc

N = 10000
E = 320000
D1 = 128
D2 = 48
NC = 2
NS = 16
NW = NC * NS
EPW = E // NW
K = 80
NWIN = EPW // K
NP = 10240
RPT = NP // NS

_MESH = dict(core_axis_name="c", subcore_axis_name="s")


_NSEM = 4


@functools.partial(
    pl.kernel,
    out_type=jax.ShapeDtypeStruct((NC * NP,), jnp.float32),
    mesh=plsc.VectorSubcoreMesh(**_MESH),
    scratch_types=[
        pltpu.VMEM((NWIN, K), jnp.int32),
        pltpu.VMEM((K,), jnp.float32),
        pltpu.VMEM((RPT,), jnp.float32),
        [pltpu.SemaphoreType.DMA] * _NSEM,
        [pltpu.SemaphoreType.DMA] * _NSEM,
        pltpu.VMEM_SHARED((NP,), jnp.float32),
    ],
)
def _sc_degree(dst_hbm, out_hbm, didx, ones_v, zeros_v, isems, sems, deg_sh):
    cid = lax.axis_index("c")
    sid = lax.axis_index("s")
    wid = cid * NS + sid

    def iload(j, _):
        for b in range(_NSEM):
            jw = j * _NSEM + b
            pltpu.async_copy(dst_hbm.at[pl.ds(wid * EPW + jw * K, K)],
                             didx.at[jw], isems[b])
        return 0
    lax.fori_loop(0, NWIN // _NSEM, iload, 0)
    for b in range(NWIN % _NSEM):
        jw = NWIN - 1 - b
        pltpu.async_copy(dst_hbm.at[pl.ds(wid * EPW + jw * K, K)],
                         didx.at[jw], isems[b])

    def fill(i, _):
        zeros_v[pl.ds(i * 16, 16)] = jnp.zeros((16,), jnp.float32)
        return 0
    lax.fori_loop(0, RPT // 16, fill, 0)
    for i in range(K // 16):
        ones_v[pl.ds(i * 16, 16)] = jnp.full((16,), 1.0, jnp.float32)

    pltpu.sync_copy(zeros_v, deg_sh.at[pl.ds(sid * RPT, RPT)])

    def idrain(j, _):
        for b in range(_NSEM):
            pltpu.make_async_copy(dst_hbm.at[pl.ds(0, K)], didx.at[0],
                                  isems[b]).wait()
        return 0
    lax.fori_loop(0, NWIN // _NSEM, idrain, 0)
    for b in range(NWIN % _NSEM):
        pltpu.make_async_copy(dst_hbm.at[pl.ds(0, K)], didx.at[0],
                              isems[b]).wait()
    plsc.subcore_barrier()

    def body(j, _):
        for b in range(_NSEM):
            pltpu.async_copy(ones_v, deg_sh.at[didx.at[j * _NSEM + b]],
                             sems[b], add=True)
        return 0
    lax.fori_loop(0, NWIN // _NSEM, body, 0)
    for b in range(NWIN % _NSEM):
        pltpu.async_copy(ones_v, deg_sh.at[didx.at[NWIN - 1 - b]],
                         sems[b], add=True)

    def drain(j, _):
        for b in range(_NSEM):
            pltpu.make_async_copy(ones_v, deg_sh.at[didx.at[0]],
                                  sems[b]).wait()
        return 0
    lax.fori_loop(0, NWIN // _NSEM, drain, 0)
    for b in range(NWIN % _NSEM):
        pltpu.make_async_copy(ones_v, deg_sh.at[didx.at[0]], sems[b]).wait()

    plsc.subcore_barrier()
    pltpu.sync_copy(deg_sh.at[pl.ds(sid * RPT, RPT)],
                    out_hbm.at[pl.ds(cid * NP + sid * RPT, RPT)])


def _make_sc_aggregate(D, NBUF, zero_buf, stage_table):
    scratch = [
        pltpu.VMEM((EPW,), jnp.int32),
        pltpu.VMEM((NBUF, K), jnp.int32),
        [pltpu.VMEM((K, D), jnp.float32)] * NBUF,
        [pltpu.SemaphoreType.DMA] * NBUF,
        [pltpu.SemaphoreType.DMA] * NBUF,
        [pltpu.SemaphoreType.DMA] * NBUF,
        pltpu.VMEM_SHARED((NP, D), jnp.float32),
    ]
    if stage_table:
        scratch.append(pltpu.VMEM_SHARED((NP, D), jnp.float32))

    @functools.partial(
        pl.kernel,
        out_type=jax.ShapeDtypeStruct((NC * NP, D), jnp.float32),
        mesh=plsc.VectorSubcoreMesh(**_MESH),
        compiler_params=pltpu.CompilerParams(use_tc_tiling_on_sc=(D % 128 == 0)),
        scratch_types=scratch,
    )
    def agg(xws_hbm, src_hbm, dst_hbm, out_hbm, sidx, didx, rows, gsem,
            ssem, isem, acc_sh, *maybe_table):
        cid = lax.axis_index("c")
        sid = lax.axis_index("s")
        wid = cid * NS + sid
        table = maybe_table[0] if stage_table else xws_hbm

        def istart(j, b):
            pltpu.async_copy(dst_hbm.at[pl.ds(wid * EPW + j * K, K)],
                             didx.at[b], isem[b])

        def iwait(b):
            pltpu.make_async_copy(dst_hbm.at[pl.ds(0, K)], didx.at[b],
                                  isem[b]).wait()

        def gstart(j, b):
            pltpu.async_copy(table.at[sidx.at[pl.ds(j * K, K)]],
                             rows[b], gsem[b])

        def gwait(b):
            pltpu.make_async_copy(table.at[sidx.at[pl.ds(0, K)]],
                                  rows[b], gsem[b]).wait()

        def sstart(b):
            pltpu.async_copy(rows[b], acc_sh.at[didx.at[b]], ssem[b],
                             add=True)

        def swait(b):
            pltpu.make_async_copy(rows[b], acc_sh.at[didx.at[b]],
                                  ssem[b]).wait()

        pltpu.sync_copy(src_hbm.at[pl.ds(wid * EPW, EPW)], sidx)
        for b in range(NBUF):
            istart(b, b)
        if stage_table:
            @pl.when(sid < 10)
            def _():
                pltpu.sync_copy(xws_hbm.at[pl.ds(sid * 1000, 1000)],
                                maybe_table[0].at[pl.ds(sid * 1000, 1000)])
        else:
            for b in range(NBUF):
                if b != zero_buf:
                    gstart(b, b)

        def fill_row(i, _):
            def fill_lane(c, _):
                rows[zero_buf][i, pl.ds(c * 16, 16)] = (
                    jnp.zeros((16,), jnp.float32))
                return 0
            lax.fori_loop(0, D // 16, fill_lane, 0)
            return 0
        lax.fori_loop(0, K, fill_row, 0)
        for i in range(RPT // K):
            pltpu.sync_copy(rows[zero_buf],
                            acc_sh.at[pl.ds(sid * RPT + i * K, K)])
        plsc.subcore_barrier()
        if stage_table:
            for b in range(NBUF):
                if b != zero_buf:
                    gstart(b, b)
        gstart(zero_buf, zero_buf)

        def group(g, _):
            for b in range(NBUF):
                j = g * NBUF + b

                @pl.when(j < NWIN)
                def _():
                    gwait(b)
                    iwait(b)
                    sstart(b)

                    @pl.when(j + NBUF < NWIN)
                    def _():
                        swait(b)
                        gstart(j + NBUF, b)
                        istart(j + NBUF, b)
            return 0
        lax.fori_loop(0, (NWIN + NBUF - 1) // NBUF, group, 0)

        for b in range(NBUF):
            swait(b)

        plsc.subcore_barrier()
        pltpu.sync_copy(acc_sh.at[pl.ds(sid * RPT, RPT)],
                        out_hbm.at[pl.ds(cid * NP + sid * RPT, RPT)])
    return agg


_sc_agg128 = _make_sc_aggregate(D1, 3, 2, False)
_sc_agg48 = _make_sc_aggregate(D2, 4, 3, False)


_DR = NP // 128


def _node_blocks():
    out = []
    for r in range((N + 127) // 128):
        n = min(128, N - r * 128)
        out.append((slice(r * 128, r * 128 + n), n, r))
    return out


def _tc_pre_body(edge_ref, x_ref, w1_ref, src_ref, dst_ref, xw_ref):
    e = edge_ref[...]
    src_ref[...] = e[0]
    dst_ref[...] = e[1]
    xw_ref[...] = jnp.dot(x_ref[...], w1_ref[...],
                          preferred_element_type=jnp.float32)


_tc_pre = pl.pallas_call(
    _tc_pre_body,
    out_shape=(jax.ShapeDtypeStruct((E,), jnp.int32),
               jax.ShapeDtypeStruct((E,), jnp.int32),
               jax.ShapeDtypeStruct((N, D1), jnp.float32)),
)


def _dt_cols(degp):
    return lax.rsqrt(degp[:_DR] + degp[_DR:] + 1.0).T


def _tc_scale_body(degp_ref, xw_ref, xws_ref):
    dt = _dt_cols(degp_ref[...])
    xw = xw_ref[...]
    for rows, n, r in _node_blocks():
        xws_ref[rows, :] = xw[rows, :] * dt[:n, r:r + 1]


_tc_scale = pl.pallas_call(
    _tc_scale_body,
    out_shape=jax.ShapeDtypeStruct((N, D1), jnp.float32),
)


def _tc_mid_body(acc, xws1, degp, b1, gamma, beta, w2p, xws2_ref, hs_ref):
    dt = _dt_cols(degp[...])
    s = acc[:N] + acc[NP:NP + N] + xws1[...]
    for rows, n, r in _node_blocks():
        hs_ref[rows, :] = s[rows, :] * dt[:n, r:r + 1]
    h = hs_ref[...] + b1[...]
    mean = jnp.mean(h, axis=0, keepdims=True)
    c = h - mean
    var = jnp.mean(c * c, axis=0, keepdims=True)
    hn = c * lax.rsqrt(var + 1e-5) * gamma[...] + beta[...]
    hr = jnp.maximum(hn, 0.0)
    xw2 = jnp.dot(hr, w2p[...], preferred_element_type=jnp.float32)
    for rows, n, r in _node_blocks():
        xws2_ref[rows, :] = xw2[rows, :] * dt[:n, r:r + 1]


_tc_mid = pl.pallas_call(
    _tc_mid_body,
    out_shape=jax.ShapeDtypeStruct((N, D2), jnp.float32),
    scratch_shapes=[pltpu.VMEM((N, D1), jnp.float32)],
)


def _tc_post_body(acc, xws2, degp, b2, out_ref):
    dp = degp[...]
    dt = lax.rsqrt(dp[:_DR] + dp[_DR:] + 1.0).T
    s = acc[:N] + acc[NP:NP + N] + xws2[...]
    for rows, n, r in _node_blocks():
        col = dt[:n, r:r + 1]
        out_ref[:, rows] = (s[rows, :NCLS] * col + b2[...]).T


NCLS = 40

_tc_post = pl.pallas_call(
    _tc_post_body,
    out_shape=jax.ShapeDtypeStruct((NCLS, N), jnp.float32),
)


def kernel(x, edge_index, W1, b1, bn_gamma, bn_beta, W2, b2):
    src, dst, xw1 = _tc_pre(edge_index, x, W1)

    degp = _sc_degree(dst)
    degp = degp.reshape(2 * _DR, 128)

    xws1 = _tc_scale(degp, xw1)

    acc1 = _sc_agg128(xws1, src, dst)

    w2p = jnp.pad(W2, ((0, 0), (0, D2 - W2.shape[1])))
    xws2 = _tc_mid(acc1, xws1, degp,
                   b1.reshape(1, D1), bn_gamma.reshape(1, D1),
                   bn_beta.reshape(1, D1), w2p)

    acc2 = _sc_agg48(xws2, src, dst)

    return _tc_post(acc2, xws2, degp, b2.reshape(1, NCLS)).T
